# baseline (speedup 1.0000x reference)
import functools

import jax
import jax.numpy as jnp
from jax import lax
from jax.experimental import pallas as pl
from jax.experimental.pallas import tpu as pltpu
from jax.experimental.pallas import tpu_sc as plsc

_NUM_CLASSES = 1000
_FEAT = 512
_B = 1024
_TEMP = 0.1
_ALPHA = 0.1
_BETA = 0.01
_MARGIN = 0.01
_GAMMA = 0.9
_K_HARD = 0.4
_EPS = 1e-16
_SM = _MARGIN / _TEMP
_NEG_FILL = -1e30

_C_PAD = 1024
_NC, _NS = 2, 16
_NW = _NC * _NS
_RPW = _B // _NW
_CROWS = _C_PAD // _NS

_ROW_T = 128
_NT = _B // _ROW_T
_BS_ITERS = 16


def _l2norm_body(x_ref, o_ref):
    x = x_ref[...]
    nrm = jnp.sqrt(jnp.sum(x * x, axis=1, keepdims=True))
    o_ref[...] = x / jnp.maximum(nrm, 1e-12)


def _l2norm_tc(emb):
    return pl.pallas_call(
        _l2norm_body,
        grid=(_NT,),
        in_specs=[pl.BlockSpec((_ROW_T, _FEAT), lambda i: (i, 0))],
        out_specs=pl.BlockSpec((_ROW_T, _FEAT), lambda i: (i, 0)),
        out_shape=jax.ShapeDtypeStruct((_B, _FEAT), jnp.float32),
    )(emb)


_CGW = 128
_NCG = _FEAT // _CGW
_CQ = 256
_ROWCHUNK = 128


def _sc_body(en_hbm, labels_hbm, sums_hbm, lab_v, chunk_v, acc_v):
    c = lax.axis_index("c")
    s = lax.axis_index("s")
    wid = s * _NC + c

    @pl.when(wid < 16)
    def _active():
        cg = wid % _NCG
        ch = wid // _NCG
        pltpu.sync_copy(labels_hbm, lab_v)

        def _zero(r, _):
            for f in range(_CGW // 16):
                acc_v[r, pl.ds(f * 16, 16)] = jnp.zeros((16,), jnp.float32)
            return 0
        lax.fori_loop(0, _CQ, _zero, 0)

        cbase = ch * _CQ
        for ck in range(_B // _ROWCHUNK):
            pltpu.sync_copy(
                en_hbm.at[pl.ds(ck * _ROWCHUNK, _ROWCHUNK),
                          pl.ds(cg * _CGW, _CGW)], chunk_v)

            def _accum(j, _, ck=ck):
                lab16 = lab_v[pl.ds(ck * _ROWCHUNK + j * 16, 16)] - cbase
                for jj in range(16):
                    lbl = lab16[jj]

                    @pl.when((lbl >= 0) & (lbl < _CQ))
                    def _add(lbl=lbl, j=j, jj=jj):
                        for f in range(_CGW // 16):
                            plsc.addupdate(
                                acc_v.at[lbl, pl.ds(f * 16, 16)],
                                chunk_v[j * 16 + jj, pl.ds(f * 16, 16)])
                return 0
            lax.fori_loop(0, _ROWCHUNK // 16, _accum, 0)

        for q in range(4):
            rows = _CQ if q < 3 else _NUM_CLASSES - 3 * _CQ

            @pl.when(ch == q)
            def _out(q=q, rows=rows):
                pltpu.sync_copy(
                    acc_v.at[pl.ds(0, rows)],
                    sums_hbm.at[pl.ds(q * _CQ, rows), pl.ds(cg * _CGW, _CGW)])


def _center_partials(en, labels):
    f = pl.kernel(
        _sc_body,
        out_type=jax.ShapeDtypeStruct((_NUM_CLASSES, _FEAT), jnp.float32),
        mesh=plsc.VectorSubcoreMesh(
            core_axis_name="c", subcore_axis_name="s",
            num_cores=_NC, num_subcores=_NS),
        scratch_types=[
            pltpu.VMEM((_B,), jnp.int32),
            pltpu.VMEM((_ROWCHUNK, _CGW), jnp.float32),
            pltpu.VMEM((_CQ, _CGW), jnp.float32),
        ],
    )
    return f(en, labels)


def _main_body(en_t_ref, en_ref, lg_ref, lrow_ref, lcol_ref,
               out_ref, cneg_ref, sacc_ref):
    i = pl.program_id(0)

    @pl.when(i == 0)
    def _init():
        for t in range(6):
            sacc_ref[t] = 0.0

    inv_t = jnp.float32(1.0 / _TEMP)
    lrow = lrow_ref[...]
    lcol = lcol_ref[...]

    en_t = en_t_ref[...]
    cs = lax.dot_general(en_t, en_ref[...], (((1,), (1,)), ((), ())),
                         preferred_element_type=jnp.float32) * inv_t
    same = lcol == lrow
    rowid = i * _ROW_T + lax.broadcasted_iota(jnp.int32, (_ROW_T, _B), 0)
    colid = lax.broadcasted_iota(jnp.int32, (_ROW_T, _B), 1)
    upper = colid > rowid
    pos = same & upper
    neg = (~same) & upper
    sacc_ref[0] += jnp.sum(pos.astype(jnp.float32))
    sacc_ref[1] += jnp.sum(jnp.where(pos, jnp.maximum(1.0 - cs, 0.0), 0.0))
    sacc_ref[2] += jnp.sum(neg.astype(jnp.float32))
    sacc_ref[3] += jnp.sum(jnp.where(neg, jnp.maximum(cs - _SM, 0.0), 0.0))
    sacc_ref[5] += jnp.sum((neg & (cs > _SM)).astype(jnp.float32))
    cneg_ref[pl.ds(i * _ROW_T, _ROW_T), :] = jnp.where(neg, cs, _NEG_FILL)

    lg = lg_ref[...] * inv_t
    mx = jnp.max(lg, axis=1, keepdims=True)
    lse = jnp.log(jnp.sum(jnp.exp(lg - mx), axis=1, keepdims=True)) + mx
    cid = lax.broadcasted_iota(jnp.int32, (_ROW_T, _NUM_CLASSES), 1)
    lab = jnp.sum(jnp.where(cid == lcol, lg, 0.0), axis=1, keepdims=True)
    sacc_ref[4] += jnp.sum(lse - lab)

    @pl.when(i == _NT - 1)
    def _tail():
        def count_gt(t):
            acc = jnp.float32(0.0)
            for bi in range(_NT):
                for bj in range(bi, _NT):
                    blk = cneg_ref[bi * _ROW_T:(bi + 1) * _ROW_T,
                                   bj * _ROW_T:(bj + 1) * _ROW_T]
                    acc += jnp.sum((blk > t).astype(jnp.float32))
            return acc

        m = 2.0 * sacc_ref[5]
        kf = jnp.maximum(1.0, jnp.floor(jnp.float32(_K_HARD) * m))

        def bs(_, carry):
            lo, hi = carry
            mid = 0.5 * (lo + hi)
            c = 2.0 * count_gt(mid)
            take = c >= kf
            return (jnp.where(take, mid, lo), jnp.where(take, hi, mid))
        lo, _hi = lax.fori_loop(
            0, _BS_ITERS, bs, (jnp.float32(_SM), jnp.float32(10.5)))

        s_gt = jnp.float32(0.0)
        c_gt = jnp.float32(0.0)
        for bi in range(_NT):
            for bj in range(bi, _NT):
                blk = cneg_ref[bi * _ROW_T:(bi + 1) * _ROW_T,
                               bj * _ROW_T:(bj + 1) * _ROW_T]
                gt = blk > lo
                s_gt += jnp.sum(jnp.where(gt, blk, 0.0))
                c_gt += jnp.sum(gt.astype(jnp.float32))
        topk_sum = 2.0 * s_gt - (2.0 * c_gt - kf) * lo
        loss_neg_hard = topk_sum / kf - _SM
        loss_neg_fb = sacc_ref[3] / sacc_ref[2]
        loss_neg = jnp.where(m > 0.0, loss_neg_hard, loss_neg_fb)
        loss_pos = sacc_ref[1] / sacc_ref[0]
        loss_cos = jnp.maximum(loss_pos + loss_neg, _EPS)
        loss_ce = sacc_ref[4] / _B
        partial = loss_ce + _ALPHA * loss_cos
        out_ref[...] = jnp.broadcast_to(partial, (1, 1))


def _main(en, logits, lrow, lcol):
    return pl.pallas_call(
        _main_body,
        grid=(_NT,),
        in_specs=[
            pl.BlockSpec((_ROW_T, _FEAT), lambda i: (i, 0)),
            pl.BlockSpec((_B, _FEAT), lambda i: (0, 0)),
            pl.BlockSpec((_ROW_T, _NUM_CLASSES), lambda i: (i, 0)),
            pl.BlockSpec((1, _B), lambda i: (0, 0)),
            pl.BlockSpec((_ROW_T, 1), lambda i: (i, 0)),
        ],
        out_specs=pl.BlockSpec((1, 1), lambda i: (0, 0)),
        out_shape=jax.ShapeDtypeStruct((1, 1), jnp.float32),
        scratch_shapes=[
            pltpu.VMEM((_B, _B), jnp.float32),
            pltpu.SMEM((8,), jnp.float32),
        ],
    )(en, en, logits, lrow, lcol)


def _finish_body(part_ref, lrow_ref, ctr_ref, psums_ref, out_ref):
    lrow = lrow_ref[...]

    tot = jnp.float32(0.0)
    for j in range(_NUM_CLASSES // _ROW_T + 1):
        rows = _ROW_T if j < _NUM_CLASSES // _ROW_T \
            else _NUM_CLASSES % _ROW_T
        sm_ = psums_ref[j * _ROW_T:j * _ROW_T + rows, :]
        cls = j * _ROW_T + lax.broadcasted_iota(jnp.int32, (rows, _B), 0)
        cnt = jnp.sum((lrow == cls).astype(jnp.float32), axis=1,
                      keepdims=True)
        ctr = ctr_ref[j * _ROW_T:j * _ROW_T + rows, :]
        newc = sm_ / (cnt + _EPS)
        upd = jnp.where(cnt > 0.0, _GAMMA * ctr + (1.0 - _GAMMA) * newc, ctr)
        nrm = jnp.maximum(
            jnp.sqrt(jnp.sum(upd * upd, axis=1, keepdims=True)), 1e-12)
        dot = jnp.sum(upd * sm_, axis=1, keepdims=True) / nrm
        tot += jnp.sum(dot)
    loss_center = jnp.maximum(1.0 - tot / (_B * _TEMP), _EPS)
    out_ref[...] = part_ref[...] + _BETA * loss_center


def _finish(partial, lrow, centers, psums):
    return pl.pallas_call(
        _finish_body,
        out_shape=jax.ShapeDtypeStruct((1, 1), jnp.float32),
    )(partial, lrow, centers, psums)


def kernel(emb, logits, labels, centers):
    en = _l2norm_tc(emb)
    psums = _center_partials(en, labels)
    lrow = labels.reshape(1, _B)
    lcol = labels.reshape(_B, 1)
    partial = _main(en, logits, lrow, lcol)
    out = _finish(partial, lrow, centers, psums)
    return out[0, 0]


# speedup vs baseline: 1.3276x; 1.3276x over previous
import functools

import jax
import jax.numpy as jnp
from jax import lax
from jax.experimental import pallas as pl
from jax.experimental.pallas import tpu as pltpu
from jax.experimental.pallas import tpu_sc as plsc

_NUM_CLASSES = 1000
_FEAT = 512
_B = 1024
_TEMP = 0.1
_ALPHA = 0.1
_BETA = 0.01
_MARGIN = 0.01
_GAMMA = 0.9
_K_HARD = 0.4
_EPS = 1e-16
_SM = _MARGIN / _TEMP
_NEG_FILL = -1e30

_C_PAD = 1024
_NC, _NS = 2, 16
_NW = _NC * _NS
_RPW = _B // _NW
_CROWS = _C_PAD // _NS

_ROW_T = 128
_NT = _B // _ROW_T
_BS_ITERS = 16


def _l2norm_body(x_ref, o_ref):
    x = x_ref[...]
    nrm = jnp.sqrt(jnp.sum(x * x, axis=1, keepdims=True))
    o_ref[...] = x / jnp.maximum(nrm, 1e-12)


def _l2norm_tc(emb):
    return pl.pallas_call(
        _l2norm_body,
        grid=(_NT,),
        in_specs=[pl.BlockSpec((_ROW_T, _FEAT), lambda i: (i, 0))],
        out_specs=pl.BlockSpec((_ROW_T, _FEAT), lambda i: (i, 0)),
        out_shape=jax.ShapeDtypeStruct((_B, _FEAT), jnp.float32),
    )(emb)


_FPW = _FEAT // _NW


def _sc_body(en_hbm, labels_hbm, sums_hbm, lab_v, rows_v, acc_v):
    c = lax.axis_index("c")
    s = lax.axis_index("s")
    wid = s * _NC + c
    fs = wid * _FPW

    pltpu.sync_copy(labels_hbm, lab_v)
    pltpu.sync_copy(en_hbm.at[:, pl.ds(fs, _FPW)], rows_v)

    z16 = jnp.zeros((_FPW,), jnp.float32)

    def _zero(r, _):
        acc_v[r, :] = z16
        return 0
    lax.fori_loop(0, _C_PAD, _zero, 0)

    def _accum(rc, _):
        base = rc * 16
        lab16 = lab_v[pl.ds(base, 16)]
        for j in range(16):
            plsc.addupdate(acc_v.at[lab16[j]], rows_v[base + j, :])
        return 0
    lax.fori_loop(0, _B // 16, _accum, 0)

    pltpu.sync_copy(acc_v, sums_hbm.at[:, pl.ds(fs, _FPW)])


def _center_partials(en, labels):
    f = pl.kernel(
        _sc_body,
        out_type=jax.ShapeDtypeStruct((_C_PAD, _FEAT), jnp.float32),
        mesh=plsc.VectorSubcoreMesh(
            core_axis_name="c", subcore_axis_name="s",
            num_cores=_NC, num_subcores=_NS),
        scratch_types=[
            pltpu.VMEM((_B,), jnp.int32),
            pltpu.VMEM((_B, _FPW), jnp.float32),
            pltpu.VMEM((_C_PAD, _FPW), jnp.float32),
        ],
        compiler_params=pltpu.CompilerParams(use_tc_tiling_on_sc=False),
    )
    return f(en, labels)


def _main_body(en_ref, lg_ref, lrow_ref, lcol_ref,
               out_ref, cneg_ref, sacc_ref):
    i = pl.program_id(0)

    @pl.when(i == 0)
    def _init():
        for t in range(6):
            sacc_ref[t] = 0.0

    inv_t = jnp.float32(1.0 / _TEMP)
    lrow = lrow_ref[...]
    lcol = lcol_ref[...]

    en_t = en_ref[pl.ds(i * _ROW_T, _ROW_T), :]
    cs = lax.dot_general(en_t, en_ref[...], (((1,), (1,)), ((), ())),
                         preferred_element_type=jnp.float32) * inv_t
    same = lcol == lrow
    rowid = i * _ROW_T + lax.broadcasted_iota(jnp.int32, (_ROW_T, _B), 0)
    colid = lax.broadcasted_iota(jnp.int32, (_ROW_T, _B), 1)
    upper = colid > rowid
    pos = same & upper
    neg = (~same) & upper
    sacc_ref[0] += jnp.sum(pos.astype(jnp.float32))
    sacc_ref[1] += jnp.sum(jnp.where(pos, jnp.maximum(1.0 - cs, 0.0), 0.0))
    sacc_ref[2] += jnp.sum(neg.astype(jnp.float32))
    sacc_ref[3] += jnp.sum(jnp.where(neg, jnp.maximum(cs - _SM, 0.0), 0.0))
    sacc_ref[5] += jnp.sum((neg & (cs > _SM)).astype(jnp.float32))
    cneg_ref[pl.ds(i * _ROW_T, _ROW_T), :] = jnp.where(neg, cs, _NEG_FILL)

    lg = lg_ref[...] * inv_t
    mx = jnp.max(lg, axis=1, keepdims=True)
    lse = jnp.log(jnp.sum(jnp.exp(lg - mx), axis=1, keepdims=True)) + mx
    cid = lax.broadcasted_iota(jnp.int32, (_ROW_T, _NUM_CLASSES), 1)
    lab = jnp.sum(jnp.where(cid == lcol, lg, 0.0), axis=1, keepdims=True)
    sacc_ref[4] += jnp.sum(lse - lab)

    @pl.when(i == _NT - 1)
    def _tail():
        def count_gt(t):
            acc = jnp.float32(0.0)
            for bi in range(_NT):
                for bj in range(bi, _NT):
                    blk = cneg_ref[bi * _ROW_T:(bi + 1) * _ROW_T,
                                   bj * _ROW_T:(bj + 1) * _ROW_T]
                    acc += jnp.sum((blk > t).astype(jnp.float32))
            return acc

        m = 2.0 * sacc_ref[5]
        kf = jnp.maximum(1.0, jnp.floor(jnp.float32(_K_HARD) * m))

        def bs(_, carry):
            lo, hi = carry
            mid = 0.5 * (lo + hi)
            c = 2.0 * count_gt(mid)
            take = c >= kf
            return (jnp.where(take, mid, lo), jnp.where(take, hi, mid))
        lo, _hi = lax.fori_loop(
            0, _BS_ITERS, bs, (jnp.float32(_SM), jnp.float32(10.5)))

        s_gt = jnp.float32(0.0)
        c_gt = jnp.float32(0.0)
        for bi in range(_NT):
            for bj in range(bi, _NT):
                blk = cneg_ref[bi * _ROW_T:(bi + 1) * _ROW_T,
                               bj * _ROW_T:(bj + 1) * _ROW_T]
                gt = blk > lo
                s_gt += jnp.sum(jnp.where(gt, blk, 0.0))
                c_gt += jnp.sum(gt.astype(jnp.float32))
        topk_sum = 2.0 * s_gt - (2.0 * c_gt - kf) * lo
        loss_neg_hard = topk_sum / kf - _SM
        loss_neg_fb = sacc_ref[3] / sacc_ref[2]
        loss_neg = jnp.where(m > 0.0, loss_neg_hard, loss_neg_fb)
        loss_pos = sacc_ref[1] / sacc_ref[0]
        loss_cos = jnp.maximum(loss_pos + loss_neg, _EPS)
        loss_ce = sacc_ref[4] / _B
        partial = loss_ce + _ALPHA * loss_cos
        out_ref[...] = jnp.broadcast_to(partial, (1, 1))


def _main(en, logits, lrow, lcol):
    return pl.pallas_call(
        _main_body,
        grid=(_NT,),
        in_specs=[
            pl.BlockSpec((_B, _FEAT), lambda i: (0, 0)),
            pl.BlockSpec((_ROW_T, _NUM_CLASSES), lambda i: (i, 0)),
            pl.BlockSpec((1, _B), lambda i: (0, 0)),
            pl.BlockSpec((_ROW_T, 1), lambda i: (i, 0)),
        ],
        out_specs=pl.BlockSpec((1, 1), lambda i: (0, 0)),
        out_shape=jax.ShapeDtypeStruct((1, 1), jnp.float32),
        scratch_shapes=[
            pltpu.VMEM((_B, _B), jnp.float32),
            pltpu.SMEM((8,), jnp.float32),
        ],
    )(en, logits, lrow, lcol)


def _finish_body(part_ref, lrow_ref, ctr_ref, psums_ref, out_ref):
    lrow = lrow_ref[...]

    tot = jnp.float32(0.0)
    for j in range(_NUM_CLASSES // _ROW_T + 1):
        rows = _ROW_T if j < _NUM_CLASSES // _ROW_T \
            else _NUM_CLASSES % _ROW_T
        sm_ = psums_ref[j * _ROW_T:j * _ROW_T + rows, :]
        cls = j * _ROW_T + lax.broadcasted_iota(jnp.int32, (rows, _B), 0)
        cnt = jnp.sum((lrow == cls).astype(jnp.float32), axis=1,
                      keepdims=True)
        ctr = ctr_ref[j * _ROW_T:j * _ROW_T + rows, :]
        newc = sm_ / (cnt + _EPS)
        upd = jnp.where(cnt > 0.0, _GAMMA * ctr + (1.0 - _GAMMA) * newc, ctr)
        nrm = jnp.maximum(
            jnp.sqrt(jnp.sum(upd * upd, axis=1, keepdims=True)), 1e-12)
        dot = jnp.sum(upd * sm_, axis=1, keepdims=True) / nrm
        tot += jnp.sum(dot)
    loss_center = jnp.maximum(1.0 - tot / (_B * _TEMP), _EPS)
    out_ref[...] = part_ref[...] + _BETA * loss_center


def _finish(partial, lrow, centers, psums):
    return pl.pallas_call(
        _finish_body,
        out_shape=jax.ShapeDtypeStruct((1, 1), jnp.float32),
    )(partial, lrow, centers, psums)


def kernel(emb, logits, labels, centers):
    en = _l2norm_tc(emb)
    psums = _center_partials(en, labels)
    lrow = labels.reshape(1, _B)
    lcol = labels.reshape(_B, 1)
    partial = _main(en, logits, lrow, lcol)
    out = _finish(partial, lrow, centers, psums)
    return out[0, 0]


# speedup vs baseline: 1.8890x; 1.4229x over previous
import jax
import jax.numpy as jnp
from jax import lax
from jax.experimental import pallas as pl
from jax.experimental.pallas import tpu as pltpu
from jax.experimental.pallas import tpu_sc as plsc

_NUM_CLASSES = 1000
_FEAT = 512
_B = 1024
_TEMP = 0.1
_ALPHA = 0.1
_BETA = 0.01
_MARGIN = 0.01
_GAMMA = 0.9
_K_HARD = 0.4
_EPS = 1e-16
_SM = _MARGIN / _TEMP
_NEG_FILL = -1e30

_C_PAD = 1024
_NC, _NS = 2, 16
_NW = _NC * _NS
_RPW = _B // _NW
_CROWS = _C_PAD // _NS

_ROW_T = 128
_NT = _B // _ROW_T
_BS_ITERS = 10
_N_UPPER = _B * (_B - 1) // 2


def _l2norm_body(x_ref, of_ref):
    x = x_ref[...]
    nrm = jnp.sqrt(jnp.sum(x * x, axis=1, keepdims=True))
    en = x / jnp.maximum(nrm, 1e-12)
    of_ref[...] = en.reshape(x.shape[0] * _FEAT)


def _l2norm_tc(emb):
    rows = _B // 2
    return pl.pallas_call(
        _l2norm_body,
        grid=(2,),
        in_specs=[pl.BlockSpec((rows, _FEAT), lambda i: (i, 0))],
        out_specs=pl.BlockSpec((rows * _FEAT,), lambda i: (i,)),
        out_shape=jax.ShapeDtypeStruct((_B * _FEAT,), jnp.float32),
    )(emb)


_FPW = _FEAT // _NW


def _sc_body(en_hbm, labels_hbm, sums_hbm, lab_v, rows_v, acc_v):
    c = lax.axis_index("c")
    s = lax.axis_index("s")
    wid = s * _NC + c
    fs = wid * _FPW

    pltpu.sync_copy(labels_hbm, lab_v)
    pltpu.sync_copy(en_hbm.at[:, pl.ds(fs, _FPW)], rows_v)

    z16 = jnp.zeros((_FPW,), jnp.float32)

    def _zero(r, _):
        acc_v[r, :] = z16
        return 0
    lax.fori_loop(0, _C_PAD, _zero, 0)

    def _accum(rc, _):
        base = rc * 16
        lab16 = lab_v[pl.ds(base, 16)]
        for j in range(16):
            plsc.addupdate(acc_v.at[lab16[j]], rows_v[base + j, :])
        return 0
    lax.fori_loop(0, _B // 16, _accum, 0)

    pltpu.sync_copy(acc_v, sums_hbm.at[:, pl.ds(fs, _FPW)])


def _center_partials(en, labels):
    f = pl.kernel(
        _sc_body,
        out_type=jax.ShapeDtypeStruct((_C_PAD, _FEAT), jnp.float32),
        mesh=plsc.VectorSubcoreMesh(
            core_axis_name="c", subcore_axis_name="s",
            num_cores=_NC, num_subcores=_NS),
        scratch_types=[
            pltpu.VMEM((_B,), jnp.int32),
            pltpu.VMEM((_B, _FPW), jnp.float32),
            pltpu.VMEM((_C_PAD, _FPW), jnp.float32),
        ],
        compiler_params=pltpu.CompilerParams(use_tc_tiling_on_sc=False),
    )
    return f(en, labels)


def _main_body(emb_ref, lgt_ref, lrow_ref, out_ref, cnt_ref,
               en_s, cneg_ref, sacc_ref):
    i = pl.program_id(0)

    @pl.when(i == 0)
    def _init():
        for t in range(6):
            sacc_ref[t] = 0.0
        for j in range(_NT):
            x = emb_ref[j * _ROW_T:(j + 1) * _ROW_T, :]
            nrm = jnp.sqrt(jnp.sum(x * x, axis=1, keepdims=True))
            en_s[j * _ROW_T:(j + 1) * _ROW_T, :] = x / jnp.maximum(nrm, 1e-12)

    inv_t = jnp.float32(1.0 / _TEMP)
    lrow = lrow_ref[...]
    ltile = lrow_ref[:, pl.ds(i * _ROW_T, _ROW_T)]
    lcol = jnp.transpose(ltile)

    en_t = en_s[pl.ds(i * _ROW_T, _ROW_T), :]
    cs = lax.dot_general(en_t, en_s[...], (((1,), (1,)), ((), ())),
                         preferred_element_type=jnp.float32) * inv_t
    same = lcol == lrow
    rowid = i * _ROW_T + lax.broadcasted_iota(jnp.int32, (_ROW_T, _B), 0)
    colid = lax.broadcasted_iota(jnp.int32, (_ROW_T, _B), 1)
    upper = colid > rowid
    pos = same & upper
    sacc_ref[0] += jnp.sum(pos.astype(jnp.float32))
    sacc_ref[1] += jnp.sum(jnp.where(pos, jnp.maximum(1.0 - cs, 0.0), 0.0))
    cneg_ref[pl.ds(i * _ROW_T, _ROW_T), :] = jnp.where(
        upper & (~same), cs, _NEG_FILL)

    lgt = lgt_ref[...] * inv_t
    mx = jnp.max(lgt, axis=0, keepdims=True)
    lse = jnp.log(jnp.sum(jnp.exp(lgt - mx), axis=0, keepdims=True)) + mx
    rid = lax.broadcasted_iota(jnp.int32, (_NUM_CLASSES, _ROW_T), 0)
    lab = jnp.sum(jnp.where(rid == ltile, lgt, 0.0), axis=0, keepdims=True)
    sacc_ref[4] += jnp.sum(lse - lab)

    @pl.when(i == _NT - 1)
    def _tail():
        def scan_gt(t):
            cnt = jnp.zeros((_ROW_T, _ROW_T), jnp.float32)
            tot = jnp.zeros((_ROW_T, _ROW_T), jnp.float32)
            for bi in range(_NT):
                for bj in range(bi, _NT):
                    blk = cneg_ref[bi * _ROW_T:(bi + 1) * _ROW_T,
                                   bj * _ROW_T:(bj + 1) * _ROW_T]
                    gt = blk > t
                    cnt += gt.astype(jnp.float32)
                    tot += jnp.where(gt, blk, 0.0)
            return jnp.sum(cnt), jnp.sum(tot)

        def count_gt(t):
            acc = jnp.zeros((_ROW_T, _ROW_T), jnp.float32)
            for bi in range(_NT):
                for bj in range(bi, _NT):
                    blk = cneg_ref[bi * _ROW_T:(bi + 1) * _ROW_T,
                                   bj * _ROW_T:(bj + 1) * _ROW_T]
                    acc += (blk > t).astype(jnp.float32)
            return jnp.sum(acc)

        m_up, s_sm = scan_gt(jnp.float32(_SM))
        m = 2.0 * m_up
        kf = jnp.maximum(1.0, jnp.floor(jnp.float32(_K_HARD) * m))

        def bs(_, carry):
            lo, hi = carry
            mid = 0.5 * (lo + hi)
            c = 2.0 * count_gt(mid)
            take = c >= kf
            return (jnp.where(take, mid, lo), jnp.where(take, hi, mid))
        lo, _hi = lax.fori_loop(
            0, _BS_ITERS, bs, (jnp.float32(_SM), jnp.float32(10.5)))

        c_gt, s_gt = scan_gt(lo)
        topk_sum = 2.0 * s_gt - (2.0 * c_gt - kf) * lo
        loss_neg_hard = topk_sum / kf - _SM
        neg_up = _N_UPPER - sacc_ref[0]
        loss_neg_fb = (s_sm - m_up * _SM) / neg_up
        loss_neg = jnp.where(m > 0.0, loss_neg_hard, loss_neg_fb)
        loss_pos = sacc_ref[1] / sacc_ref[0]
        loss_cos = jnp.maximum(loss_pos + loss_neg, _EPS)
        loss_ce = sacc_ref[4] / _B
        partial = loss_ce + _ALPHA * loss_cos
        out_ref[...] = jnp.broadcast_to(partial, (1, 1))

        for j in range(_NT):
            cls = j * _ROW_T + lax.broadcasted_iota(
                jnp.int32, (_ROW_T, _B), 0)
            cnt_ref[pl.ds(j * _ROW_T, _ROW_T), :] = jnp.sum(
                (lrow == cls).astype(jnp.float32), axis=1, keepdims=True)


def _main(emb, logits_t, lrow):
    return pl.pallas_call(
        _main_body,
        grid=(_NT,),
        in_specs=[
            pl.BlockSpec((_B, _FEAT), lambda i: (0, 0)),
            pl.BlockSpec((_NUM_CLASSES, _ROW_T), lambda i: (0, i)),
            pl.BlockSpec((1, _B), lambda i: (0, 0)),
        ],
        out_specs=(
            pl.BlockSpec((1, 1), lambda i: (0, 0)),
            pl.BlockSpec((_B, 1), lambda i: (0, 0)),
        ),
        out_shape=(
            jax.ShapeDtypeStruct((1, 1), jnp.float32),
            jax.ShapeDtypeStruct((_B, 1), jnp.float32),
        ),
        scratch_shapes=[
            pltpu.VMEM((_B, _FEAT), jnp.float32),
            pltpu.VMEM((_B, _B), jnp.float32),
            pltpu.SMEM((8,), jnp.float32),
        ],
    )(emb, logits_t, lrow)


def _finish_body(part_ref, cnt_ref, ctr_ref, psums_ref, out_ref):
    tot = jnp.float32(0.0)
    for j in range(_NUM_CLASSES // _ROW_T + 1):
        rows = _ROW_T if j < _NUM_CLASSES // _ROW_T \
            else _NUM_CLASSES % _ROW_T
        sm_ = psums_ref[j * _ROW_T * _FEAT:
                        (j * _ROW_T + rows) * _FEAT].reshape(rows, _FEAT)
        cnt = cnt_ref[j * _ROW_T:j * _ROW_T + rows, :]
        ctr = ctr_ref[j * _ROW_T:j * _ROW_T + rows, :]
        newc = sm_ / (cnt + _EPS)
        upd = jnp.where(cnt > 0.0, _GAMMA * ctr + (1.0 - _GAMMA) * newc, ctr)
        nrm = jnp.maximum(
            jnp.sqrt(jnp.sum(upd * upd, axis=1, keepdims=True)), 1e-12)
        dot = jnp.sum(upd * sm_, axis=1, keepdims=True) / nrm
        tot += jnp.sum(dot)
    loss_center = jnp.maximum(1.0 - tot / (_B * _TEMP), _EPS)
    out_ref[...] = part_ref[...] + _BETA * loss_center


def _finish(partial, cnt, centers, psums):
    return pl.pallas_call(
        _finish_body,
        out_shape=jax.ShapeDtypeStruct((1, 1), jnp.float32),
    )(partial, cnt, centers, psums)


def kernel(emb, logits, labels, centers):
    en_flat = _l2norm_tc(emb)
    psums = _center_partials(en_flat.reshape(_B, _FEAT), labels)
    lrow = labels.reshape(1, _B)
    partial, cnt = _main(emb, logits.T, lrow)
    out = _finish(partial, cnt, centers, psums.reshape(_C_PAD * _FEAT))
    return out[0, 0]
